# baseline (speedup 1.0000x reference)
import functools

import jax
import jax.numpy as jnp
from jax import lax
from jax.experimental import pallas as pl
from jax.experimental.pallas import tpu as pltpu
from jax.experimental.pallas import tpu_sc as plsc

B = 64
S = 2048
H = 64
NC = 2
NS = 16
NW = NC * NS
SBLK = 128
NSB = S // SBLK
BPW = B // (NW // NSB)
LANES = 16
G = SBLK // LANES


def _emb_body(x_hbm, wt2_hbm, posT_hbm, out_hbm,
              xbuf, idx_v, pbufT, gbuf, obuf, gsem):
  wid = lax.axis_index("s") * NC + lax.axis_index("c")
  sb = wid % NSB
  s0 = sb * SBLK
  b0 = (wid // NSB) * BPW

  pltpu.sync_copy(posT_hbm.at[:, pl.ds(s0, SBLK)], pbufT)

  rows = [lax.iota(jnp.int32, LANES) + g * LANES for g in range(G)]

  @pl.loop(0, BPW)
  def _chunk(i):
    b = b0 + i
    pltpu.sync_copy(x_hbm.at[b, pl.ds(s0, SBLK)], xbuf)
    pars = []
    for g in range(G):
      v = xbuf[pl.ds(g * LANES, LANES)]
      idx_v[pl.ds(g * LANES, LANES)] = lax.shift_right_logical(v, 1)
      pars.append((v & 1) * H)
    pltpu.async_copy(wt2_hbm.at[idx_v], gbuf, gsem).wait()
    @pl.loop(0, H, unroll=4)
    def _h(h):
      for g in range(G):
        sl = pl.ds(g * LANES, LANES)
        vec = plsc.load_gather(gbuf, [rows[g], pars[g] + h])
        obuf[h, sl] = vec + pbufT[h, sl]
    pltpu.sync_copy(obuf, out_hbm.at[b, :, pl.ds(s0, SBLK)])


@jax.jit
def _emb(x, wt2, posT):
  mesh = plsc.VectorSubcoreMesh(
      core_axis_name="c", subcore_axis_name="s", num_cores=NC, num_subcores=NS
  )
  return pl.kernel(
      _emb_body,
      out_type=jax.ShapeDtypeStruct((B, H, S), jnp.float32),
      mesh=mesh,
      scratch_types=[
          pltpu.VMEM((SBLK,), jnp.int32),
          pltpu.VMEM((SBLK,), jnp.int32),
          pltpu.VMEM((H, SBLK), jnp.float32),
          pltpu.VMEM((SBLK, SBLK), jnp.float32),
          pltpu.VMEM((H, SBLK), jnp.float32),
          pltpu.SemaphoreType.DMA,
      ],
      compiler_params=pltpu.CompilerParams(needs_layout_passes=False),
  )(x, wt2, posT)


def kernel(x, word_table, pos_table):
  x = x.astype(jnp.int32)
  wt2 = word_table.reshape(500000, 128)
  posT = jnp.swapaxes(pos_table, 0, 1)
  out = _emb(x, wt2, posT)
  return jnp.swapaxes(out, 1, 2)


# speedup vs baseline: 1.0773x; 1.0773x over previous
import functools

import jax
import jax.numpy as jnp
from jax import lax
from jax.experimental import pallas as pl
from jax.experimental.pallas import tpu as pltpu
from jax.experimental.pallas import tpu_sc as plsc

B = 64
S = 2048
H = 64
HP = 128
NC = 2
NS = 16
NW = NC * NS
SBLK = 128
NSB = S // SBLK
BPW = B // (NW // NSB)
LANES = 16
G = SBLK // LANES


def _emb_body(x_hbm, wtp_hbm, posT_hbm, out_hbm,
              xbuf, pbufT, gbuf, obuf, gsem):
  wid = lax.axis_index("s") * NC + lax.axis_index("c")
  sb = wid % NSB
  s0 = sb * SBLK
  b0 = (wid // NSB) * BPW

  pltpu.sync_copy(posT_hbm.at[:, pl.ds(s0, SBLK)], pbufT)

  rows = [lax.iota(jnp.int32, LANES) + g * LANES for g in range(G)]

  @pl.loop(0, BPW)
  def _chunk(i):
    b = b0 + i
    pltpu.sync_copy(x_hbm.at[b, pl.ds(s0, SBLK)], xbuf)
    pltpu.async_copy(wtp_hbm.at[xbuf], gbuf, gsem).wait()
    @pl.loop(0, H, unroll=8)
    def _h(h):
      hvec = jnp.full((LANES,), 0, jnp.int32) + h
      for g in range(G):
        sl = pl.ds(g * LANES, LANES)
        vec = plsc.load_gather(gbuf, [rows[g], hvec])
        obuf[h, sl] = vec + pbufT[h, sl]
    pltpu.sync_copy(obuf, out_hbm.at[b, :, pl.ds(s0, SBLK)])


@jax.jit
def _emb(x, wtp, posT):
  mesh = plsc.VectorSubcoreMesh(
      core_axis_name="c", subcore_axis_name="s", num_cores=NC, num_subcores=NS
  )
  return pl.kernel(
      _emb_body,
      out_type=jax.ShapeDtypeStruct((B, H, S), jnp.float32),
      mesh=mesh,
      scratch_types=[
          pltpu.VMEM((SBLK,), jnp.int32),
          pltpu.VMEM((H, SBLK), jnp.float32),
          pltpu.VMEM((SBLK, HP), jnp.float32),
          pltpu.VMEM((H, SBLK), jnp.float32),
          pltpu.SemaphoreType.DMA,
      ],
      compiler_params=pltpu.CompilerParams(needs_layout_passes=False),
  )(x, wtp, posT)


def kernel(x, word_table, pos_table):
  x = x.astype(jnp.int32)
  wtp = jnp.pad(word_table, ((0, 0), (0, HP - H)))
  posT = jnp.swapaxes(pos_table, 0, 1)
  out = _emb(x, wtp, posT)
  return jnp.swapaxes(out, 1, 2)


# speedup vs baseline: 2.1029x; 1.9520x over previous
import functools

import jax
import jax.numpy as jnp
from jax import lax
from jax.experimental import pallas as pl
from jax.experimental.pallas import tpu as pltpu
from jax.experimental.pallas import tpu_sc as plsc

B = 64
S = 2048
H = 64
NC = 2
NS = 16
NW = NC * NS
SBLK = 128
NSB = S // SBLK
BPW = B // (NW // NSB)
LANES = 16
G = SBLK // LANES


def _emb_body(x_hbm, wt_hbm, posT_hbm, out_hbm,
              xbuf, pbufT, gbuf, obuf, gsem):
  wid = lax.axis_index("s") * NC + lax.axis_index("c")
  sb = wid % NSB
  s0 = sb * SBLK
  b0 = (wid // NSB) * BPW

  pltpu.sync_copy(posT_hbm.at[:, pl.ds(s0, SBLK)], pbufT)

  rows = [lax.iota(jnp.int32, LANES) + g * LANES for g in range(G)]

  @pl.loop(0, BPW)
  def _chunk(i):
    b = b0 + i
    pltpu.sync_copy(x_hbm.at[b, pl.ds(s0, SBLK)], xbuf)

    for g in range(G):
      vg = xbuf[pl.ds(g * LANES, LANES)]
      for k in range(LANES):
        pltpu.async_copy(wt_hbm.at[vg[k]], gbuf.at[g * LANES + k], gsem)
    pltpu.make_async_copy(wt_hbm.at[pl.ds(0, SBLK)], gbuf, gsem).wait()

    @functools.partial(plsc.parallel_loop, 0, H, unroll=8)
    def _h(h):
      hvec = jnp.full((LANES,), 0, jnp.int32) + h
      for g in range(G):
        sl = pl.ds(g * LANES, LANES)
        vec = plsc.load_gather(gbuf, [rows[g], hvec])
        obuf[h, sl] = vec + pbufT[h, sl]

    pltpu.sync_copy(obuf, out_hbm.at[b, :, pl.ds(s0, SBLK)])


@jax.jit
def _emb(x, word_table, posT):
  mesh = plsc.VectorSubcoreMesh(
      core_axis_name="c", subcore_axis_name="s", num_cores=NC, num_subcores=NS
  )
  return pl.kernel(
      _emb_body,
      out_type=jax.ShapeDtypeStruct((B, H, S), jnp.float32),
      mesh=mesh,
      scratch_types=[
          pltpu.VMEM((SBLK,), jnp.int32),
          pltpu.VMEM((H, SBLK), jnp.float32),
          pltpu.VMEM((SBLK, H), jnp.float32),
          pltpu.VMEM((H, SBLK), jnp.float32),
          pltpu.SemaphoreType.DMA,
      ],
      compiler_params=pltpu.CompilerParams(needs_layout_passes=False),
  )(x, word_table, posT)


def kernel(x, word_table, pos_table):
  x = x.astype(jnp.int32)
  posT = jnp.swapaxes(pos_table, 0, 1)
  out = _emb(x, word_table, posT)
  return jnp.swapaxes(out, 1, 2)
